# baseline (speedup 1.0000x reference)
import functools

import jax
import jax.numpy as jnp
from jax import lax
from jax.experimental import pallas as pl
from jax.experimental.pallas import tpu as pltpu
from jax.experimental.pallas import tpu_sc as plsc

_ALPHA = 0.1
_BETA = 0.5
_K = 9
_EPS = 1e-5
_INF = float("inf")


def _tc_score_idx(x_ref, idx_ref):
    i = pl.program_id(0)
    A = x_ref[0]
    C, N = A.shape
    G = jax.lax.dot_general(A, A, (((0,), (0,)), ((), ())),
                            preferred_element_type=jnp.float32)
    sq = jnp.sum(A * A, axis=0, keepdims=True)
    scoreT = jnp.transpose(sq) - 2.0 * G
    riota = jax.lax.broadcasted_iota(jnp.int32, (N, N), 0)
    rows = []
    for _ in range(_K):
        rmin = jnp.min(scoreT, axis=0, keepdims=True)
        hit = scoreT == rmin
        idxk = jnp.min(jnp.where(hit, riota, N), axis=0, keepdims=True)
        scoreT = jnp.where(riota == idxk, _INF, scoreT)
        rows.append(idxk)
    rows += [rows[0]] * (16 - _K)
    idx_ref[...] = jnp.concatenate(rows, axis=0) + i * N


def _sc_gather_body(tok_ref, idx_ref, nm_ref, idx_v, acc_v, sem):
    wid = lax.axis_index("s") * 2 + lax.axis_index("c")
    base = wid * 128
    pltpu.sync_copy(idx_ref.at[:, pl.ds(base, 128)], idx_v)
    pltpu.async_copy(tok_ref.at[idx_v.at[0]], acc_v, sem).wait()
    cps = [pltpu.async_copy(tok_ref.at[idx_v.at[j]], acc_v, sem, add=True)
           for j in range(1, _K)]
    for cp in cps:
        cp.wait()
    pltpu.sync_copy(acc_v, nm_ref.at[pl.ds(base, 128), :])


def _tc_finish(nm_ref, x_ref, x0_ref, w_ref, b_ref, g_ref, bb_ref,
               out_ref, pre_ref):
    B, C, N = x_ref.shape
    i = pl.program_id(0)
    nmT = jnp.transpose(nm_ref[i])[:C]
    h = ((1.0 - _ALPHA) / _K) * nmT + _ALPHA * x0_ref[i]
    lin = jnp.dot(w_ref[...], h, preferred_element_type=jnp.float32) + b_ref[...]
    pre_ref[i] = (1.0 - _BETA) * h + _BETA * lin

    @pl.when(i == B - 1)
    def _bn():
        s1 = pre_ref[0]
        for j in range(1, B):
            s1 = s1 + pre_ref[j]
        mean = jnp.sum(s1, axis=1, keepdims=True) * (1.0 / (B * N))
        s2 = jnp.zeros((C, 1), jnp.float32)
        for j in range(B):
            d = pre_ref[j] - mean
            s2 = s2 + jnp.sum(d * d, axis=1, keepdims=True)
        var = s2 * (1.0 / (B * N))
        scale = g_ref[...] * jax.lax.rsqrt(var + _EPS)
        shift = bb_ref[...] - mean * scale
        for j in range(B):
            out_ref[j] = jnp.maximum(pre_ref[j] * scale + shift + x_ref[j], 0.0)


def _sc_gather(tok_nc, idx):
    BN, Cp = tok_nc.shape
    mesh = plsc.VectorSubcoreMesh(core_axis_name="c", subcore_axis_name="s")
    return pl.kernel(
        _sc_gather_body,
        mesh=mesh,
        out_type=jax.ShapeDtypeStruct((BN, Cp), jnp.float32),
        scratch_types=[
            pltpu.VMEM((16, BN // 32), jnp.int32),
            pltpu.VMEM((BN // 32, Cp), jnp.float32),
            pltpu.SemaphoreType.DMA,
        ],
    )(tok_nc, idx)


def kernel(x, x_0, Wlin, b, gamma, beta_bn):
    B, C, H, W = x.shape
    N = H * W
    x3 = x.reshape(B, C, N)
    x03 = x_0.reshape(B, C, N)
    tok_nc = jnp.pad(x3.transpose(0, 2, 1).reshape(B * N, C),
                     ((0, 0), (0, 128 - C)))
    b2 = b.reshape(C, 1)
    g2 = gamma.reshape(C, 1)
    bb2 = beta_bn.reshape(C, 1)

    idx = pl.pallas_call(
        _tc_score_idx,
        grid=(B,),
        in_specs=[pl.BlockSpec((1, C, N), lambda i: (i, 0, 0))],
        out_specs=pl.BlockSpec((16, N), lambda i: (0, i)),
        out_shape=jax.ShapeDtypeStruct((16, B * N), jnp.int32),
    )(x3)

    nm = _sc_gather(tok_nc, idx)

    full3 = pl.BlockSpec((B, C, N), lambda i: (0, 0, 0))
    col = pl.BlockSpec((C, 1), lambda i: (0, 0))
    out = pl.pallas_call(
        _tc_finish,
        grid=(B,),
        in_specs=[
            pl.BlockSpec((B, N, 128), lambda i: (0, 0, 0)),
            full3,
            full3,
            pl.BlockSpec((C, C), lambda i: (0, 0)),
            col, col, col,
        ],
        out_specs=full3,
        out_shape=jax.ShapeDtypeStruct((B, C, N), jnp.float32),
        scratch_shapes=[pltpu.VMEM((B, C, N), jnp.float32)],
    )(nm.reshape(B, N, 128), x3, x03, Wlin, b2, g2, bb2)
    return out.reshape(B, C, H, W)


# speedup vs baseline: 1.0252x; 1.0252x over previous
import functools

import jax
import jax.numpy as jnp
from jax import lax
from jax.experimental import pallas as pl
from jax.experimental.pallas import tpu as pltpu
from jax.experimental.pallas import tpu_sc as plsc

_ALPHA = 0.1
_BETA = 0.5
_K = 9
_EPS = 1e-5
_INF = float("inf")


def _tc_score_idx(x_ref, idx_ref):
    i = pl.program_id(0)
    A = x_ref[0]
    C, N = A.shape
    G = jax.lax.dot_general(A, A, (((0,), (0,)), ((), ())),
                            preferred_element_type=jnp.float32)
    sq = jnp.sum(A * A, axis=0, keepdims=True)
    scoreT = jnp.transpose(sq) - 2.0 * G
    riota = jax.lax.broadcasted_iota(jnp.int32, (N, N), 0)
    rows = []
    for _ in range(_K):
        rmin = jnp.min(scoreT, axis=0, keepdims=True)
        hit = scoreT == rmin
        idxk = jnp.min(jnp.where(hit, riota, N), axis=0, keepdims=True)
        scoreT = jnp.where(hit, _INF, scoreT)
        rows.append(idxk)
    rows += [rows[0]] * (16 - _K)
    idx_ref[...] = jnp.concatenate(rows, axis=0) + i * N


def _sc_gather_body(tok_ref, idx_ref, zero_ref, nm_ref, idx_v, acc_v, sem):
    wid = lax.axis_index("s") * 2 + lax.axis_index("c")
    base = wid * 128
    z = pltpu.async_copy(zero_ref, acc_v, sem)
    pltpu.sync_copy(idx_ref.at[:, pl.ds(base, 128)], idx_v)
    z.wait()
    cps = [pltpu.async_copy(tok_ref.at[idx_v.at[j]], acc_v, sem, add=True)
           for j in range(_K)]
    for cp in cps:
        cp.wait()
    pltpu.sync_copy(acc_v, nm_ref.at[pl.ds(base, 128), :])


def _tc_finish(nm_ref, x_ref, x0_ref, w_ref, b_ref, g_ref, bb_ref,
               out_ref, pre_ref):
    B, C, N = x_ref.shape
    i = pl.program_id(0)
    nmT = jnp.transpose(nm_ref[i])[:C]
    h = ((1.0 - _ALPHA) / _K) * nmT + _ALPHA * x0_ref[i]
    lin = jnp.dot(w_ref[...], h, preferred_element_type=jnp.float32) + b_ref[...]
    pre_ref[i] = (1.0 - _BETA) * h + _BETA * lin

    @pl.when(i == B - 1)
    def _bn():
        s1 = pre_ref[0]
        for j in range(1, B):
            s1 = s1 + pre_ref[j]
        mean = jnp.sum(s1, axis=1, keepdims=True) * (1.0 / (B * N))
        s2 = jnp.zeros((C, 1), jnp.float32)
        for j in range(B):
            d = pre_ref[j] - mean
            s2 = s2 + jnp.sum(d * d, axis=1, keepdims=True)
        var = s2 * (1.0 / (B * N))
        scale = g_ref[...] * jax.lax.rsqrt(var + _EPS)
        shift = bb_ref[...] - mean * scale
        for j in range(B):
            out_ref[j] = jnp.maximum(pre_ref[j] * scale + shift + x_ref[j], 0.0)


def _sc_gather(tok_nc, idx, zeros):
    BN, Cp = tok_nc.shape
    mesh = plsc.VectorSubcoreMesh(core_axis_name="c", subcore_axis_name="s")
    return pl.kernel(
        _sc_gather_body,
        mesh=mesh,
        out_type=jax.ShapeDtypeStruct((BN, Cp), jnp.float32),
        scratch_types=[
            pltpu.VMEM((16, BN // 32), jnp.int32),
            pltpu.VMEM((BN // 32, Cp), jnp.float32),
            pltpu.SemaphoreType.DMA,
        ],
    )(tok_nc, idx, zeros)


def kernel(x, x_0, Wlin, b, gamma, beta_bn):
    B, C, H, W = x.shape
    N = H * W
    x3 = x.reshape(B, C, N)
    x03 = x_0.reshape(B, C, N)
    tok_nc = jnp.pad(x3.transpose(0, 2, 1).reshape(B * N, C),
                     ((0, 0), (0, 128 - C)))
    b2 = b.reshape(C, 1)
    g2 = gamma.reshape(C, 1)
    bb2 = beta_bn.reshape(C, 1)

    idx = pl.pallas_call(
        _tc_score_idx,
        grid=(B,),
        in_specs=[pl.BlockSpec((1, C, N), lambda i: (i, 0, 0))],
        out_specs=pl.BlockSpec((16, N), lambda i: (0, i)),
        out_shape=jax.ShapeDtypeStruct((16, B * N), jnp.int32),
    )(x3)

    zeros = jnp.zeros((N // 8, 128), jnp.float32)
    nm = _sc_gather(tok_nc, idx, zeros)

    full3 = pl.BlockSpec((B, C, N), lambda i: (0, 0, 0))
    col = pl.BlockSpec((C, 1), lambda i: (0, 0))
    out = pl.pallas_call(
        _tc_finish,
        grid=(B,),
        in_specs=[
            pl.BlockSpec((B, N, 128), lambda i: (0, 0, 0)),
            full3,
            full3,
            pl.BlockSpec((C, C), lambda i: (0, 0)),
            col, col, col,
        ],
        out_specs=full3,
        out_shape=jax.ShapeDtypeStruct((B, C, N), jnp.float32),
        scratch_shapes=[pltpu.VMEM((B, C, N), jnp.float32)],
    )(nm.reshape(B, N, 128), x3, x03, Wlin, b2, g2, bb2)
    return out.reshape(B, C, H, W)


# speedup vs baseline: 1.1439x; 1.1158x over previous
import jax
import jax.numpy as jnp
from jax import lax
from jax.experimental import pallas as pl
from jax.experimental.pallas import tpu as pltpu
from jax.experimental.pallas import tpu_sc as plsc

_ALPHA = 0.1
_BETA = 0.5
_K = 9
_EPS = 1e-5
_INF = float("inf")


def _tc_score_idx(x_ref, idx_ref, tok_ref):
    i = pl.program_id(0)
    A = x_ref[0]
    C, N = A.shape
    G = jax.lax.dot_general(A, A, (((0,), (0,)), ((), ())),
                            preferred_element_type=jnp.float32)
    sq = jnp.sum(A * A, axis=0, keepdims=True)
    score = sq - 2.0 * G
    ciota = jax.lax.broadcasted_iota(jnp.int32, (N, N), 1).astype(jnp.float32)
    cols = []
    for _ in range(_K):
        rmin = jnp.min(score, axis=1, keepdims=True)
        hit = score == rmin
        idxk = jnp.min(jnp.where(hit, ciota, float(N)), axis=1, keepdims=True)
        score = jnp.where(hit, _INF, score)
        cols.append(idxk)
    cols += [cols[0]] * (16 - _K)
    idx_ref[...] = (jnp.transpose(jnp.concatenate(cols, axis=1)).astype(jnp.int32)
                    + i * N)
    tok_ref[...] = jnp.concatenate(
        [jnp.transpose(A), jnp.zeros((N, 128 - C), jnp.float32)],
        axis=1)


def _sc_gather_body(tok_ref, idx_ref, zero_ref, nm_ref, idx_v, acc_v, sem):
    wid = lax.axis_index("s") * 2 + lax.axis_index("c")
    base = wid * 128
    z = pltpu.async_copy(zero_ref, acc_v, sem)
    pltpu.sync_copy(idx_ref.at[:, pl.ds(base, 128)], idx_v)
    z.wait()
    cps = [pltpu.async_copy(tok_ref.at[idx_v.at[j]], acc_v, sem, add=True)
           for j in range(_K)]
    for cp in cps:
        cp.wait()
    pltpu.sync_copy(acc_v, nm_ref.at[pl.ds(base, 128), :])


def _tc_finish(nm_ref, x_ref, x0_ref, w_ref, b_ref, g_ref, bb_ref,
               out_ref, pre_ref):
    B, C, N = x_ref.shape
    i = pl.program_id(0)
    nmT = jnp.transpose(nm_ref[0])[:C]
    h = ((1.0 - _ALPHA) / _K) * nmT + _ALPHA * x0_ref[0]
    lin = jnp.dot(w_ref[...], h, preferred_element_type=jnp.float32) + b_ref[...]
    pre_ref[i] = (1.0 - _BETA) * h + _BETA * lin

    @pl.when(i == B - 1)
    def _bn():
        s1 = pre_ref[0]
        for j in range(1, B):
            s1 = s1 + pre_ref[j]
        mean = jnp.sum(s1, axis=1, keepdims=True) * (1.0 / (B * N))
        s2 = jnp.zeros((C, 1), jnp.float32)
        for j in range(B):
            d = pre_ref[j] - mean
            s2 = s2 + jnp.sum(d * d, axis=1, keepdims=True)
        var = s2 * (1.0 / (B * N))
        scale = g_ref[...] * jax.lax.rsqrt(var + _EPS)
        shift = bb_ref[...] - mean * scale
        for j in range(B):
            out_ref[j] = jnp.maximum(pre_ref[j] * scale + shift + x_ref[j], 0.0)


def _sc_gather(tok_nc, idx, zeros):
    BN, Cp = tok_nc.shape
    mesh = plsc.VectorSubcoreMesh(core_axis_name="c", subcore_axis_name="s")
    return pl.kernel(
        _sc_gather_body,
        mesh=mesh,
        out_type=jax.ShapeDtypeStruct((BN, Cp), jnp.float32),
        scratch_types=[
            pltpu.VMEM((16, BN // 32), jnp.int32),
            pltpu.VMEM((BN // 32, Cp), jnp.float32),
            pltpu.SemaphoreType.DMA,
        ],
    )(tok_nc, idx, zeros)


def kernel(x, x_0, Wlin, b, gamma, beta_bn):
    B, C, H, W = x.shape
    N = H * W
    x3 = x.reshape(B, C, N)
    x03 = x_0.reshape(B, C, N)
    b2 = b.reshape(C, 1)
    g2 = gamma.reshape(C, 1)
    bb2 = beta_bn.reshape(C, 1)

    idx, tok_nc = pl.pallas_call(
        _tc_score_idx,
        grid=(B,),
        in_specs=[pl.BlockSpec((1, C, N), lambda i: (i, 0, 0))],
        out_specs=[
            pl.BlockSpec((16, N), lambda i: (0, i)),
            pl.BlockSpec((N, 128), lambda i: (i, 0)),
        ],
        out_shape=[
            jax.ShapeDtypeStruct((16, B * N), jnp.int32),
            jax.ShapeDtypeStruct((B * N, 128), jnp.float32),
        ],
    )(x3)

    zeros = jnp.zeros((N // 8, 128), jnp.float32)
    nm = _sc_gather(tok_nc, idx, zeros)

    full3 = pl.BlockSpec((B, C, N), lambda i: (0, 0, 0))
    col = pl.BlockSpec((C, 1), lambda i: (0, 0))
    out = pl.pallas_call(
        _tc_finish,
        grid=(B,),
        in_specs=[
            pl.BlockSpec((1, N, 128), lambda i: (i, 0, 0)),
            full3,
            pl.BlockSpec((1, C, N), lambda i: (i, 0, 0)),
            pl.BlockSpec((C, C), lambda i: (0, 0)),
            col, col, col,
        ],
        out_specs=full3,
        out_shape=jax.ShapeDtypeStruct((B, C, N), jnp.float32),
        scratch_shapes=[pltpu.VMEM((B, C, N), jnp.float32)],
    )(nm.reshape(B, N, 128), x3, x03, Wlin, b2, g2, bb2)
    return out.reshape(B, C, H, W)
